# baseline (speedup 1.0000x reference)
import functools

import jax
import jax.numpy as jnp
import numpy as np
from jax.experimental import pallas as pl
from jax.experimental.pallas import tpu as pltpu

_G = 256
_SIGMA = 2.0
_TARGET = 1.0


def _build_smooth_matrix():
    k_size = int(6 * _SIGMA) | 1
    x = np.arange(k_size, dtype=np.float32) - k_size // 2
    k1 = np.exp(-(x ** 2) / (2.0 * _SIGMA ** 2))
    w = (k1 / k1.sum()).astype(np.float64)
    pad = k_size // 2
    s = np.zeros((_G, _G), dtype=np.float64)
    for t in range(k_size):
        off = t - pad
        for g in range(_G):
            i = g + off
            if i < 0:
                i = -i
            elif i >= _G:
                i = 2 * _G - 2 - i
            s[g, i] += w[t]
    return s.astype(np.float32)


_SMOOTH = _build_smooth_matrix()


def _split(x):
    hi = x.astype(jnp.bfloat16)
    lo = (x - hi.astype(jnp.float32)).astype(jnp.bfloat16)
    return hi, lo


def _dot3_tb(a, b):
    ah, al = _split(a)
    bh, bl = _split(b)
    dims = (((1,), (1,)), ((), ()))
    d = jax.lax.dot_general(al, bh, dims, preferred_element_type=jnp.float32)
    d += jax.lax.dot_general(ah, bl, dims, preferred_element_type=jnp.float32)
    d += jax.lax.dot_general(ah, bh, dims, preferred_element_type=jnp.float32)
    return d


def _body(px_ref, py_ref, ax_ref, ay_ref, sh_ref, sl_ref, den_ref, loss_ref):
    g = _G
    gx = (px_ref[0] + 1.0) * ((g - 1) / 2.0)
    gy = (py_ref[0] + 1.0) * ((g - 1) / 2.0)
    v = gx.shape[1]
    coords = jax.lax.broadcasted_iota(jnp.int32, (g, v), 0).astype(jnp.float32)
    x_in = jax.nn.sigmoid(ax_ref[0] - 2.0 * jnp.abs(coords - gx))
    y_in = jax.nn.sigmoid(ay_ref[0] - 2.0 * jnp.abs(coords - gy))
    d = _dot3_tb(y_in, x_in)
    sh, sl = sh_ref[...], sl_ref[...]
    dh, dl = _split(d)
    dims_n = (((1,), (0,)), ((), ()))
    t = jax.lax.dot_general(sl, dh, dims_n, preferred_element_type=jnp.float32)
    t += jax.lax.dot_general(sh, dl, dims_n, preferred_element_type=jnp.float32)
    t += jax.lax.dot_general(sh, dh, dims_n, preferred_element_type=jnp.float32)
    th, tl = _split(t)
    dims_t = (((1,), (1,)), ((), ()))
    out = jax.lax.dot_general(tl, sh, dims_t, preferred_element_type=jnp.float32)
    out += jax.lax.dot_general(th, sl, dims_t, preferred_element_type=jnp.float32)
    out += jax.lax.dot_general(th, sh, dims_t, preferred_element_type=jnp.float32)
    den_ref[...] = out[None]
    ov = jnp.maximum(out - _TARGET, 0.0)
    part = jnp.sum(ov * ov, axis=0, keepdims=True)
    loss_ref[...] = (part[:, :128] + part[:, 128:]).reshape(1, 1, 128)


@jax.jit
def kernel(positions, sizes, macro_mask):
    b, v, _ = positions.shape
    g = _G
    px = positions[:, :, 0].reshape(b, 1, v)
    py = positions[:, :, 1].reshape(b, 1, v)
    mask = macro_mask
    ax = (sizes[:, 0] * (g / 2.0)).reshape(1, 1, v)
    ay = jnp.where(mask, sizes[:, 1] * (g / 2.0), -1e9).reshape(1, 1, v)
    smooth = jnp.asarray(_SMOOTH)
    smooth_hi = smooth.astype(jnp.bfloat16)
    smooth_lo = (smooth - smooth_hi.astype(jnp.float32)).astype(jnp.bfloat16)

    den, loss_part = pl.pallas_call(
        _body,
        grid=(b,),
        in_specs=[
            pl.BlockSpec((1, 1, v), lambda i: (i, 0, 0)),
            pl.BlockSpec((1, 1, v), lambda i: (i, 0, 0)),
            pl.BlockSpec((1, 1, v), lambda i: (0, 0, 0)),
            pl.BlockSpec((1, 1, v), lambda i: (0, 0, 0)),
            pl.BlockSpec((g, g), lambda i: (0, 0)),
            pl.BlockSpec((g, g), lambda i: (0, 0)),
        ],
        out_specs=[
            pl.BlockSpec((1, g, g), lambda i: (i, 0, 0)),
            pl.BlockSpec((1, 1, 128), lambda i: (i, 0, 0)),
        ],
        out_shape=[
            jax.ShapeDtypeStruct((b, g, g), jnp.float32),
            jax.ShapeDtypeStruct((b, 1, 128), jnp.float32),
        ],
        compiler_params=pltpu.CompilerParams(
            dimension_semantics=("parallel",),
        ),
    )(px, py, ax, ay, smooth_hi, smooth_lo)

    density = den.reshape(b, 1, g, g)
    overflow_loss = jnp.sum(loss_part) / (b * g * g)
    return density, overflow_loss


# speedup vs baseline: 1.0908x; 1.0908x over previous
import functools

import jax
import jax.numpy as jnp
import numpy as np
from jax.experimental import pallas as pl
from jax.experimental.pallas import tpu as pltpu

_G = 256
_SIGMA = 2.0
_TARGET = 1.0


def _build_smooth_matrix():
    k_size = int(6 * _SIGMA) | 1
    x = np.arange(k_size, dtype=np.float32) - k_size // 2
    k1 = np.exp(-(x ** 2) / (2.0 * _SIGMA ** 2))
    w = (k1 / k1.sum()).astype(np.float64)
    pad = k_size // 2
    s = np.zeros((_G, _G), dtype=np.float64)
    for t in range(k_size):
        off = t - pad
        for g in range(_G):
            i = g + off
            if i < 0:
                i = -i
            elif i >= _G:
                i = 2 * _G - 2 - i
            s[g, i] += w[t]
    return s.astype(np.float32)


_SMOOTH = _build_smooth_matrix()


def _split(x):
    hi = x.astype(jnp.bfloat16)
    lo = (x - hi.astype(jnp.float32)).astype(jnp.bfloat16)
    return hi, lo


def _dot3_tb(a, b):
    ah, al = _split(a)
    bh, bl = _split(b)
    dims = (((1,), (1,)), ((), ()))
    d = jax.lax.dot_general(al, bh, dims, preferred_element_type=jnp.float32)
    d += jax.lax.dot_general(ah, bl, dims, preferred_element_type=jnp.float32)
    d += jax.lax.dot_general(ah, bh, dims, preferred_element_type=jnp.float32)
    return d


_L2E = 1.4426950408889634


def _body(px_ref, py_ref, ax_ref, ay_ref, sh_ref, sl_ref, den_ref, loss_ref):
    g = _G
    gx = (px_ref[0] + 1.0) * ((g - 1) * _L2E)
    gy = (py_ref[0] + 1.0) * ((g - 1) * _L2E)
    v = gx.shape[1]
    coords = (jax.lax.broadcasted_iota(jnp.int32, (g, v), 0)
              .astype(jnp.float32) * (2.0 * _L2E))
    x_in = 1.0 / (1.0 + jnp.exp2(jnp.abs(coords - gx) - ax_ref[0]))
    y_in = 1.0 / (1.0 + jnp.exp2(jnp.abs(coords - gy) - ay_ref[0]))
    d = _dot3_tb(y_in, x_in)
    sh, sl = sh_ref[...], sl_ref[...]
    dh, dl = _split(d)
    dims_n = (((1,), (0,)), ((), ()))
    t = jax.lax.dot_general(sl, dh, dims_n, preferred_element_type=jnp.float32)
    t += jax.lax.dot_general(sh, dl, dims_n, preferred_element_type=jnp.float32)
    t += jax.lax.dot_general(sh, dh, dims_n, preferred_element_type=jnp.float32)
    th, tl = _split(t)
    dims_t = (((1,), (1,)), ((), ()))
    out = jax.lax.dot_general(tl, sh, dims_t, preferred_element_type=jnp.float32)
    out += jax.lax.dot_general(th, sl, dims_t, preferred_element_type=jnp.float32)
    out += jax.lax.dot_general(th, sh, dims_t, preferred_element_type=jnp.float32)
    den_ref[...] = out[None]
    ov = jnp.maximum(out - _TARGET, 0.0)
    part = jnp.sum(ov * ov, axis=0, keepdims=True)
    loss_ref[...] = (part[:, :128] + part[:, 128:]).reshape(1, 1, 128)


@jax.jit
def kernel(positions, sizes, macro_mask):
    b, v, _ = positions.shape
    g = _G
    px = positions[:, :, 0].reshape(b, 1, v)
    py = positions[:, :, 1].reshape(b, 1, v)
    mask = macro_mask
    ax = (sizes[:, 0] * (g / 2.0) * _L2E).reshape(1, 1, v)
    ay = jnp.where(mask, sizes[:, 1] * (g / 2.0) * _L2E,
                   -1e30).reshape(1, 1, v)
    smooth = jnp.asarray(_SMOOTH)
    smooth_hi = smooth.astype(jnp.bfloat16)
    smooth_lo = (smooth - smooth_hi.astype(jnp.float32)).astype(jnp.bfloat16)

    den, loss_part = pl.pallas_call(
        _body,
        grid=(b,),
        in_specs=[
            pl.BlockSpec((1, 1, v), lambda i: (i, 0, 0)),
            pl.BlockSpec((1, 1, v), lambda i: (i, 0, 0)),
            pl.BlockSpec((1, 1, v), lambda i: (0, 0, 0)),
            pl.BlockSpec((1, 1, v), lambda i: (0, 0, 0)),
            pl.BlockSpec((g, g), lambda i: (0, 0)),
            pl.BlockSpec((g, g), lambda i: (0, 0)),
        ],
        out_specs=[
            pl.BlockSpec((1, g, g), lambda i: (i, 0, 0)),
            pl.BlockSpec((1, 1, 128), lambda i: (i, 0, 0)),
        ],
        out_shape=[
            jax.ShapeDtypeStruct((b, g, g), jnp.float32),
            jax.ShapeDtypeStruct((b, 1, 128), jnp.float32),
        ],
        compiler_params=pltpu.CompilerParams(
            dimension_semantics=("parallel",),
        ),
    )(px, py, ax, ay, smooth_hi, smooth_lo)

    density = den.reshape(b, 1, g, g)
    overflow_loss = jnp.sum(loss_part) / (b * g * g)
    return density, overflow_loss


# speedup vs baseline: 1.3811x; 1.2661x over previous
import functools

import jax
import jax.numpy as jnp
import numpy as np
from jax.experimental import pallas as pl
from jax.experimental.pallas import tpu as pltpu

_G = 256
_SIGMA = 2.0
_TARGET = 1.0


def _build_smooth_matrix():
    k_size = int(6 * _SIGMA) | 1
    x = np.arange(k_size, dtype=np.float32) - k_size // 2
    k1 = np.exp(-(x ** 2) / (2.0 * _SIGMA ** 2))
    w = (k1 / k1.sum()).astype(np.float64)
    pad = k_size // 2
    s = np.zeros((_G, _G), dtype=np.float64)
    for t in range(k_size):
        off = t - pad
        for g in range(_G):
            i = g + off
            if i < 0:
                i = -i
            elif i >= _G:
                i = 2 * _G - 2 - i
            s[g, i] += w[t]
    return s.astype(np.float32)


_SMOOTH = _build_smooth_matrix()


def _split(x):
    hi = x.astype(jnp.bfloat16)
    lo = (x - hi.astype(jnp.float32)).astype(jnp.bfloat16)
    return hi, lo


_DIMS_NN = (((1,), (0,)), ((), ()))
_DIMS_NT = (((1,), (1,)), ((), ()))


def _dot2(a_hi, a_lo, b, dims):
    d = jax.lax.dot_general(a_lo, b, dims, preferred_element_type=jnp.float32)
    d += jax.lax.dot_general(a_hi, b, dims, preferred_element_type=jnp.float32)
    return d


_L2E = 1.4426950408889634


def _body(px_ref, py_ref, ax_ref, ay_ref, sh_ref, sl_ref, den_ref, loss_ref):
    g = _G
    gx = (px_ref[0] + 1.0) * ((g - 1) * _L2E)
    gy = (py_ref[0] + 1.0) * ((g - 1) * _L2E)
    v = gx.shape[1]
    coords = (jax.lax.broadcasted_iota(jnp.int32, (g, v), 0)
              .astype(jnp.float32) * (2.0 * _L2E))
    x_in = 1.0 / (1.0 + jnp.exp2(jnp.abs(coords - gx) - ax_ref[0]))
    y_in = 1.0 / (1.0 + jnp.exp2(jnp.abs(coords - gy) - ay_ref[0]))
    yh, yl = _split(y_in)
    xb = x_in.astype(jnp.bfloat16)
    d = _dot2(yh, yl, xb, _DIMS_NT)
    sh, sl = sh_ref[...], sl_ref[...]
    t = _dot2(sh, sl, d.astype(jnp.bfloat16), _DIMS_NT)
    out = _dot2(sh, sl, t.astype(jnp.bfloat16), _DIMS_NT)
    den_ref[...] = out[None]
    ov = jnp.maximum(out - _TARGET, 0.0)
    part = jnp.sum(ov * ov, axis=0, keepdims=True)
    loss_ref[...] = (part[:, :128] + part[:, 128:]).reshape(1, 1, 128)


@jax.jit
def kernel(positions, sizes, macro_mask):
    b, v, _ = positions.shape
    g = _G
    px = positions[:, :, 0].reshape(b, 1, v)
    py = positions[:, :, 1].reshape(b, 1, v)
    mask = macro_mask
    ax = (sizes[:, 0] * (g / 2.0) * _L2E).reshape(1, 1, v)
    ay = jnp.where(mask, sizes[:, 1] * (g / 2.0) * _L2E,
                   -1e30).reshape(1, 1, v)
    smooth = jnp.asarray(_SMOOTH)
    smooth_hi = smooth.astype(jnp.bfloat16)
    smooth_lo = (smooth - smooth_hi.astype(jnp.float32)).astype(jnp.bfloat16)

    den, loss_part = pl.pallas_call(
        _body,
        grid=(b,),
        in_specs=[
            pl.BlockSpec((1, 1, v), lambda i: (i, 0, 0)),
            pl.BlockSpec((1, 1, v), lambda i: (i, 0, 0)),
            pl.BlockSpec((1, 1, v), lambda i: (0, 0, 0)),
            pl.BlockSpec((1, 1, v), lambda i: (0, 0, 0)),
            pl.BlockSpec((g, g), lambda i: (0, 0)),
            pl.BlockSpec((g, g), lambda i: (0, 0)),
        ],
        out_specs=[
            pl.BlockSpec((1, g, g), lambda i: (i, 0, 0)),
            pl.BlockSpec((1, 1, 128), lambda i: (i, 0, 0)),
        ],
        out_shape=[
            jax.ShapeDtypeStruct((b, g, g), jnp.float32),
            jax.ShapeDtypeStruct((b, 1, 128), jnp.float32),
        ],
        compiler_params=pltpu.CompilerParams(
            dimension_semantics=("parallel",),
        ),
    )(px, py, ax, ay, smooth_hi, smooth_lo)

    density = den.reshape(b, 1, g, g)
    overflow_loss = jnp.sum(loss_part) / (b * g * g)
    return density, overflow_loss


# speedup vs baseline: 1.5329x; 1.1099x over previous
import functools

import jax
import jax.numpy as jnp
import numpy as np
from jax.experimental import pallas as pl
from jax.experimental.pallas import tpu as pltpu

_G = 256
_SIGMA = 2.0
_TARGET = 1.0


def _build_smooth_matrix():
    k_size = int(6 * _SIGMA) | 1
    x = np.arange(k_size, dtype=np.float32) - k_size // 2
    k1 = np.exp(-(x ** 2) / (2.0 * _SIGMA ** 2))
    w = (k1 / k1.sum()).astype(np.float64)
    pad = k_size // 2
    s = np.zeros((_G, _G), dtype=np.float64)
    for t in range(k_size):
        off = t - pad
        for g in range(_G):
            i = g + off
            if i < 0:
                i = -i
            elif i >= _G:
                i = 2 * _G - 2 - i
            s[g, i] += w[t]
    return s.astype(np.float32)


_SMOOTH = _build_smooth_matrix()


def _split(x):
    hi = x.astype(jnp.bfloat16)
    lo = (x - hi.astype(jnp.float32)).astype(jnp.bfloat16)
    return hi, lo


_DIMS_NN = (((1,), (0,)), ((), ()))
_DIMS_NT = (((1,), (1,)), ((), ()))


def _dot2(a_hi, a_lo, b, dims):
    d = jax.lax.dot_general(a_lo, b, dims, preferred_element_type=jnp.float32)
    d += jax.lax.dot_general(a_hi, b, dims, preferred_element_type=jnp.float32)
    return d


def _body(px_ref, py_ref, ax_ref, ay_ref, sqh_ref, sql_ref, sh_ref, sl_ref,
          den_ref, loss_ref):
    g = _G
    gx = (px_ref[0] + 1.0) * ((g - 1) / 2.0)
    gy = (py_ref[0] + 1.0) * ((g - 1) / 2.0)
    v = gx.shape[1]
    coords = jax.lax.broadcasted_iota(jnp.int32, (g, v), 0).astype(jnp.float32)
    x_in = 1.0 + jnp.tanh(ax_ref[0] - jnp.abs(coords - gx))
    y_in = 1.0 + jnp.tanh(ay_ref[0] - jnp.abs(coords - gy))
    yh, yl = _split(y_in)
    xb = x_in.astype(jnp.bfloat16)
    d = _dot2(yh, yl, xb, _DIMS_NT)
    t = _dot2(sqh_ref[...], sql_ref[...], d.astype(jnp.bfloat16), _DIMS_NT)
    out = _dot2(sh_ref[...], sl_ref[...], t.astype(jnp.bfloat16), _DIMS_NT)
    den_ref[...] = out[None]
    ov = jnp.maximum(out - _TARGET, 0.0)
    part = jnp.sum(ov * ov, axis=0, keepdims=True)
    loss_ref[...] = (part[:, :128] + part[:, 128:]).reshape(1, 1, 128)


@jax.jit
def kernel(positions, sizes, macro_mask):
    b, v, _ = positions.shape
    g = _G
    px = positions[:, :, 0].reshape(b, 1, v)
    py = positions[:, :, 1].reshape(b, 1, v)
    mask = macro_mask
    ax = (sizes[:, 0] * (g / 4.0)).reshape(1, 1, v)
    ay = jnp.where(mask, sizes[:, 1] * (g / 4.0), -1e30).reshape(1, 1, v)
    smooth = jnp.asarray(_SMOOTH)
    smooth_q = smooth * 0.25
    smooth_q_hi = smooth_q.astype(jnp.bfloat16)
    smooth_q_lo = (smooth_q - smooth_q_hi.astype(jnp.float32)
                   ).astype(jnp.bfloat16)
    smooth_hi = smooth.astype(jnp.bfloat16)
    smooth_lo = (smooth - smooth_hi.astype(jnp.float32)).astype(jnp.bfloat16)

    den, loss_part = pl.pallas_call(
        _body,
        grid=(b,),
        in_specs=[
            pl.BlockSpec((1, 1, v), lambda i: (i, 0, 0)),
            pl.BlockSpec((1, 1, v), lambda i: (i, 0, 0)),
            pl.BlockSpec((1, 1, v), lambda i: (0, 0, 0)),
            pl.BlockSpec((1, 1, v), lambda i: (0, 0, 0)),
            pl.BlockSpec((g, g), lambda i: (0, 0)),
            pl.BlockSpec((g, g), lambda i: (0, 0)),
            pl.BlockSpec((g, g), lambda i: (0, 0)),
            pl.BlockSpec((g, g), lambda i: (0, 0)),
        ],
        out_specs=[
            pl.BlockSpec((1, g, g), lambda i: (i, 0, 0)),
            pl.BlockSpec((1, 1, 128), lambda i: (i, 0, 0)),
        ],
        out_shape=[
            jax.ShapeDtypeStruct((b, g, g), jnp.float32),
            jax.ShapeDtypeStruct((b, 1, 128), jnp.float32),
        ],
        compiler_params=pltpu.CompilerParams(
            dimension_semantics=("parallel",),
        ),
    )(px, py, ax, ay, smooth_q_hi, smooth_q_lo, smooth_hi, smooth_lo)

    density = den.reshape(b, 1, g, g)
    overflow_loss = jnp.sum(loss_part) / (b * g * g)
    return density, overflow_loss
